# baseline (speedup 1.0000x reference)
import dataclasses
import functools

import jax
import jax.numpy as jnp
from jax import lax
from jax.experimental import pallas as pl
from jax.experimental.pallas import tpu as pltpu
from jax.experimental.pallas import tpu_sc as plsc

_BT = 2048
_NW = 32
_E = 16


def _logits_block(x_ref, w_ref, out_ref):
    out_ref[...] = jnp.dot(x_ref[...], w_ref[...],
                           preferred_element_type=jnp.float32)


def _tc_logits(x, W, chunk, n_chunks):
    T, D = x.shape
    E = W.shape[1]
    rows = T // n_chunks
    blocks = rows // _BT
    return pl.pallas_call(
        _logits_block,
        grid=(blocks,),
        in_specs=[
            pl.BlockSpec((_BT, D), lambda i: (chunk * blocks + i, 0)),
            pl.BlockSpec((D, E), lambda i: (0, 0)),
        ],
        out_specs=pl.BlockSpec((_BT, E), lambda i: (i, 0)),
        out_shape=jax.ShapeDtypeStruct((rows, E), jnp.float32),
        compiler_params=pltpu.CompilerParams(
            dimension_semantics=("parallel",),
        ),
    )(x, W)


def _sc_route(logits):
    T = logits.shape[0]
    rows = T // _NW
    mesh = plsc.VectorSubcoreMesh(core_axis_name="c", subcore_axis_name="s")
    cp = pltpu.CompilerParams()
    if "needs_layout_passes" in pltpu.CompilerParams.__dataclass_fields__:
        cp = dataclasses.replace(cp, needs_layout_passes=False)

    @functools.partial(
        pl.kernel,
        mesh=mesh,
        compiler_params=cp,
        out_type=jax.ShapeDtypeStruct((T, _E), jnp.float32),
        scratch_types=[
            pltpu.VMEM((rows, _E), jnp.float32),
            pltpu.VMEM((rows, _E), jnp.float32),
        ],
    )
    def k(lg_hbm, out_hbm, in_v, out_v):
        wid = lax.axis_index("s") * 2 + lax.axis_index("c")
        base = wid * rows
        pltpu.sync_copy(lg_hbm.at[pl.ds(base, rows)], in_v)

        lane = lax.iota(jnp.int32, _E)
        lane_tag = (_E - 1) - lane
        low_mask = jnp.full((_E,), -_E, jnp.int32)
        pos_mask = jnp.full((_E,), 0x7FFFFFFF, jnp.int32)
        int_min = jnp.full((_E,), jnp.int32(-2147483648))
        one = jnp.full((_E,), 1.0, jnp.float32)
        zero = jnp.zeros((_E,), jnp.float32)

        @plsc.parallel_loop(0, rows, unroll=8)
        def _(r):
            v = in_v[r, :]
            b = plsc.bitcast(v, jnp.int32) & low_mask
            key = (b ^ ((b >> 31) & pos_mask)) & low_mask | lane_tag
            k1 = jnp.broadcast_to(jnp.max(key), (_E,))
            eq1 = key == k1
            rest = jnp.where(eq1, int_min, key)
            k2 = jnp.broadcast_to(jnp.max(rest), (_E,))
            eq2 = rest == k2
            kc1 = k1 & low_mask
            kc2 = k2 & low_mask
            m1 = plsc.bitcast(kc1 ^ ((kc1 >> 31) & pos_mask), jnp.float32)
            m2 = plsc.bitcast(kc2 ^ ((kc2 >> 31) & pos_mask), jnp.float32)
            t = jnp.exp(m2 - m1)
            g1 = one / (one + t)
            g2 = t * g1
            out_v[r, :] = jnp.where(eq1, g1, jnp.where(eq2, g2, zero))

        pltpu.sync_copy(out_v, out_hbm.at[pl.ds(base, rows)])

    return k(logits)


def kernel(x, W):
    n_chunks = 2
    logits = [_tc_logits(x, W, c, n_chunks) for c in range(n_chunks)]
    combine = [_sc_route(lg) for lg in logits]
    return jnp.concatenate(combine, axis=0)


# speedup vs baseline: 1.7153x; 1.7153x over previous
import jax
import jax.numpy as jnp
from jax.experimental import pallas as pl
from jax.experimental.pallas import tpu as pltpu

_BT = 2048


def _router_block(x_ref, w_ref, out_ref):
    logits = jnp.dot(x_ref[...], w_ref[...],
                     preferred_element_type=jnp.float32)
    e = logits.shape[-1]
    idx = jax.lax.broadcasted_iota(jnp.int32, logits.shape, 1)
    m1 = jnp.max(logits, axis=-1, keepdims=True)
    i1 = jnp.min(jnp.where(logits >= m1, idx, e), axis=-1, keepdims=True)
    masked = jnp.where(idx == i1, -jnp.inf, logits)
    m2 = jnp.max(masked, axis=-1, keepdims=True)
    i2 = jnp.min(jnp.where(masked >= m2, idx, e), axis=-1, keepdims=True)
    t = jnp.exp(m2 - m1)
    g1 = 1.0 / (1.0 + t)
    g2 = t / (1.0 + t)
    out_ref[...] = jnp.where(idx == i1, g1,
                             jnp.where(idx == i2, g2, 0.0))


def kernel(x, W):
    T, D = x.shape
    E = W.shape[1]
    return pl.pallas_call(
        _router_block,
        grid=(T // _BT,),
        in_specs=[
            pl.BlockSpec((_BT, D), lambda i: (i, 0)),
            pl.BlockSpec((D, E), lambda i: (0, 0)),
        ],
        out_specs=pl.BlockSpec((_BT, E), lambda i: (i, 0)),
        out_shape=jax.ShapeDtypeStruct((T, E), jnp.float32),
        compiler_params=pltpu.CompilerParams(
            dimension_semantics=("parallel",),
        ),
    )(x, W)
